# baseline (speedup 1.0000x reference)
import jax
import jax.numpy as jnp
from jax import lax
from jax.experimental import pallas as pl
from jax.experimental.pallas import tpu as pltpu
from jax.experimental.pallas import tpu_sc as plsc

_NC = 2
_NS = 16
_NW = _NC * _NS
_B = 64
_XCOL = 16


def _elu(v):
    return jnp.where(v > 0, v, jnp.exp(v) - 1.0)


def _prep(h_in, Wk, Wr, b2d, n_rows, n_tab, pool):
    di, do = Wk.shape[1], Wk.shape[2]
    wf = 2 * do + _XCOL

    def body(h_ref, wk_ref, wr_ref, b_ref, f_ref, r_ref):
        hv = h_ref[...]
        if pool:
            hp = jnp.maximum(hv[:, :di], hv[:, di:])
        else:
            hp = hv
        w0 = wk_ref[0]
        wd = wk_ref[1] - w0
        zc = jnp.zeros((di, _XCOL), jnp.float32)
        wcat = jnp.concatenate([w0, zc, wd], axis=1)
        hp_pad = jnp.concatenate(
            [hp, jnp.zeros((n_tab - n_rows, di), jnp.float32)], axis=0)
        cols = lax.broadcasted_iota(jnp.int32, (n_tab, wf), 1)
        ones_col = (cols == do).astype(jnp.float32)
        f_ref[...] = jnp.dot(hp_pad, wcat,
                             preferred_element_type=jnp.float32) + ones_col
        r_ref[...] = jnp.dot(hp, wr_ref[...],
                             preferred_element_type=jnp.float32) + b_ref[...]

    return pl.pallas_call(
        body,
        out_shape=[
            jax.ShapeDtypeStruct((n_tab, wf), jnp.float32),
            jax.ShapeDtypeStruct((n_rows, do), jnp.float32),
        ],
    )(h_in, Wk, Wr, b2d)


def _sc_agg(f, srcp, dstp, up, zer, shift, n_tab, wf, do, bsz, n_real):
    with_u = up is not None
    epw = srcp.shape[0] // _NW
    nb = epw // bsz
    rpt = n_tab // _NS
    nbuf = 2 if 2 * bsz * wf * 4 <= 480000 else 1

    def body(*refs):
        i = 4 if with_u else 3
        f_h, src_h, dst_h = refs[0], refs[1], refs[2]
        u_h = refs[3] if with_u else None
        zer_h = refs[i]
        out = refs[i + 1]
        rows = refs[i + 2:i + 2 + nbuf]
        isrc = refs[i + 2 + nbuf:i + 2 + 2 * nbuf]
        idst = refs[i + 2 + 2 * nbuf:i + 2 + 3 * nbuf]
        i += 2 + 3 * nbuf
        uv = refs[i] if with_u else None
        i += 1 if with_u else 0
        acc = refs[i]
        gsem = refs[i + 1:i + 1 + nbuf]

        c = lax.axis_index("c")
        s = lax.axis_index("s")
        wid = s * _NC + c
        r0 = s * rpt
        pltpu.sync_copy(zer_h.at[pl.ds(r0, rpt)], acc.at[pl.ds(r0, rpt)])
        plsc.subcore_barrier()

        def load_idx(b, p):
            off = wid * epw + b * bsz
            pltpu.sync_copy(src_h.at[pl.ds(off, bsz)], isrc[p])
            pltpu.sync_copy(dst_h.at[pl.ds(off, bsz)], idst[p])
            if shift:
                for i2 in range(bsz // 16):
                    sl = pl.ds(i2 * 16, 16)
                    isrc[p][sl] = jnp.right_shift(isrc[p][sl], shift)
                    idst[p][sl] = jnp.right_shift(idst[p][sl], shift)

        if nbuf == 2:
            load_idx(0, 0)
            cp = pltpu.async_copy(f_h.at[isrc[0]], rows[0], gsem[0])
        else:
            def bbody1(b, carry):
                load_idx(b, 0)
                pltpu.async_copy(f_h.at[isrc[0]], rows[0], gsem[0]).wait()
                pltpu.sync_copy(rows[0], acc.at[idst[0]], add=True)
                return carry

            lax.fori_loop(0, nb, bbody1, 0)
            plsc.subcore_barrier()
            pltpu.sync_copy(acc.at[pl.ds(r0, rpt)], out.at[c, pl.ds(r0,
                                                                    rpt)])
            return

        def bbody(bi, carry):
            for p in range(2):
                b = bi * 2 + p
                q = 1 - p
                nxt = jnp.minimum(b + 1, nb - 1)
                load_idx(nxt, q)
                pltpu.async_copy(f_h.at[isrc[q]], rows[q], gsem[q])
                pltpu.make_async_copy(f_h.at[isrc[p]], rows[p],
                                      gsem[p]).wait()
                if with_u:
                    off = wid * epw + b * bsz
                    pltpu.sync_copy(u_h.at[pl.ds(off, bsz)], uv)
                    iot = lax.broadcasted_iota(jnp.int32, (16,), 0)
                    for ch in range(bsz // 16):
                        uc = uv[pl.ds(ch * 16, 16)]
                        uc = jnp.minimum(jnp.maximum(uc, 0.0), 1.0)
                        for e2 in range(16):
                            ue = jnp.sum(jnp.where(iot == e2, uc, 0.0))
                            e = ch * 16 + e2
                            for ci in range(do // 16):
                                slc = pl.ds(ci * 16, 16)
                                sld = pl.ds(do + _XCOL + ci * 16, 16)
                                rows[p][e, slc] = (rows[p][e, slc]
                                                   + ue * rows[p][e, sld])
                pltpu.sync_copy(rows[p], acc.at[idst[p]], add=True)
            return carry

        lax.fori_loop(0, nb // 2, bbody, 0)
        pltpu.make_async_copy(f_h.at[isrc[0]], rows[0], gsem[0]).wait()
        plsc.subcore_barrier()
        pltpu.sync_copy(acc.at[pl.ds(r0, rpt)], out.at[c, pl.ds(r0, rpt)])

    out_type = [jax.ShapeDtypeStruct((_NC, n_tab, wf), jnp.float32)]
    scratch = ([pltpu.VMEM((bsz, wf), jnp.float32)] * nbuf
               + [pltpu.VMEM((bsz,), jnp.int32)] * (2 * nbuf))
    if with_u:
        scratch.append(pltpu.VMEM((bsz,), jnp.float32))
    scratch += ([pltpu.VMEM_SHARED((n_tab, wf), jnp.float32)]
                + [pltpu.SemaphoreType.DMA] * nbuf)
    mesh = plsc.VectorSubcoreMesh(core_axis_name="c", subcore_axis_name="s",
                                  num_cores=_NC, num_subcores=_NS)
    fn = pl.kernel(
        body, out_type=out_type, mesh=mesh, scratch_types=scratch,
        compiler_params=pltpu.CompilerParams(needs_layout_passes=False,
                                             use_tc_tiling_on_sc=False))
    res = fn(f, srcp, dstp, up, zer) if with_u else fn(f, srcp, dstp, zer)
    if isinstance(res, (list, tuple)):
        res = res[0]
    return res


def _epi_h(p_ref, r_ref, n_rows, do, with_u):
    am = p_ref[0] + p_ref[1]
    am = am[:n_rows]
    deg = am[:, do:do + 1]
    agg = am[:, :do]
    if with_u:
        u = deg / jnp.maximum(jnp.max(deg), 1.0)
        agg = agg + u * am[:, do + _XCOL:2 * do + _XCOL]
    return _elu(agg / jnp.maximum(deg, 1.0) + r_ref[...])


def _epi_prep(part2, r2, Wk, Wr, b2d, n_rows, do, with_u, n_tab_next):
    di, do2 = Wk.shape[1], Wk.shape[2]
    wf = 2 * do + _XCOL
    wf2 = 2 * do2 + _XCOL
    n2 = n_rows // 2

    def body(p_ref, r_ref, wk_ref, wr_ref, b_ref, f_ref, r2_ref):
        am = p_ref[0] + p_ref[1]
        am = am[:n2]
        rv = r_ref[...]
        halves = []
        for o, ro in ((0, 0), (wf, do)):
            deg = am[:, o + do:o + do + 1]
            agg = am[:, o:o + do]
            if with_u:
                dsum = am[:, o + do + _XCOL:o + 2 * do + _XCOL]
                halves.append((agg, dsum, deg, rv[:, ro:ro + do]))
            else:
                halves.append((agg, None, deg, rv[:, ro:ro + do]))
        if with_u:
            maxdeg = jnp.maximum(jnp.max(halves[0][2]),
                                 jnp.max(halves[1][2]))
            maxdeg = jnp.maximum(maxdeg, 1.0)
        hs = []
        for agg, dsum, deg, rh in halves:
            if with_u:
                agg = agg + (deg / maxdeg) * dsum
            hs.append(_elu(agg / jnp.maximum(deg, 1.0) + rh))
        hp = jnp.maximum(hs[0], hs[1])
        w0 = wk_ref[0]
        wd = wk_ref[1] - w0
        zc = jnp.zeros((di, _XCOL), jnp.float32)
        wcat = jnp.concatenate([w0, zc, wd], axis=1)
        hp_pad = jnp.concatenate(
            [hp, jnp.zeros((n_tab_next - n2, di), jnp.float32)], axis=0)
        cols = lax.broadcasted_iota(jnp.int32, (n_tab_next, wf2), 1)
        ones_col = (cols == do2).astype(jnp.float32)
        f_ref[...] = jnp.dot(hp_pad, wcat,
                             preferred_element_type=jnp.float32) + ones_col
        r2_ref[...] = jnp.dot(hp, wr_ref[...],
                              preferred_element_type=jnp.float32) + b_ref[...]

    return pl.pallas_call(
        body,
        out_shape=[
            jax.ShapeDtypeStruct((n_tab_next, wf2), jnp.float32),
            jax.ShapeDtypeStruct((n2, do2), jnp.float32),
        ],
    )(part2, r2, Wk, Wr, b2d)


def _epi_head(part, r, bt, w1, b1, w2, b2, n_rows, do, n_graphs):
    def body(p_ref, r_ref, bt_ref, w1_ref, b1_ref, w2_ref, b2_ref, o_ref):
        hv = _epi_h(p_ref, r_ref, n_rows, do, with_u=True)
        btv = bt_ref[...]
        ninf = jnp.float32(-jnp.inf)
        gs = []
        for j in range(n_graphs):
            mj = jnp.where(btv == j, hv, ninf)
            gs.append(jnp.max(mj, axis=0, keepdims=True))
        g = jnp.concatenate(gs, axis=0)
        g = jnp.where(jnp.isfinite(g), g, 0.0)
        z = _elu(jnp.dot(g, w1_ref[...],
                         preferred_element_type=jnp.float32) + b1_ref[...])
        z2 = jnp.dot(z, w2_ref[...],
                     preferred_element_type=jnp.float32) + b2_ref[...]
        m = jnp.max(z2, axis=1, keepdims=True)
        lse = m + jnp.log(jnp.sum(jnp.exp(z2 - m), axis=1, keepdims=True))
        o_ref[...] = z2 - lse

    out_shape = jax.ShapeDtypeStruct((n_graphs, w2.shape[1]), jnp.float32)
    return pl.pallas_call(body, out_shape=out_shape)(part, r, bt, w1, b1,
                                                     w2, b2)


def kernel(x, edge_index, weight, batch, W1, Wr1, b1, W2, Wr2, b2, W3, Wr3,
           b3, W4, Wr4, b4, W5, Wr5, b5, fc1_W, fc1_b, fc2_W, fc2_b):
    n0 = x.shape[0]
    n_edges = edge_index.shape[1]
    n_graphs = 16

    bszs = [128, 128, 128, 80, 64]
    chunk = _NW * 128
    ne = ((n_edges + chunk - 1) // chunk) * chunk
    pad = ne - n_edges
    spread = 31
    ns = [n0 >> k for k in range(5)]
    base_pad = n0 + 16 * (spread - 1)
    nts = [((base_pad >> k) // 16 + 1) * 16 for k in range(5)]

    pidx = (n0 + 16 * (jnp.arange(pad, dtype=jnp.int32) % spread)).astype(
        jnp.int32)
    srcp = jnp.concatenate([edge_index[0], pidx])
    dstp = jnp.concatenate([edge_index[1], pidx])
    up = jnp.concatenate([weight[:, 0], jnp.zeros((pad,), jnp.float32)])

    layers = [(W1, Wr1, b1), (W2, Wr2, b2), (W3, Wr3, b3), (W4, Wr4, b4),
              (W5, Wr5, b5)]
    dos = [layers[k][0].shape[2] for k in range(5)]

    f, r = _prep(x, W1, Wr1, b1.reshape(1, -1), ns[0], nts[0], pool=False)
    for k in range(5):
        do = dos[k]
        wf = 2 * do + _XCOL
        zer = jnp.zeros((nts[k], wf), jnp.float32)
        part = _sc_agg(f, srcp, dstp, up if k == 0 else None, zer, k,
                       nts[k], wf, do, bszs[k], ns[k])
        if k < 4:
            Wk, Wr, bb = layers[k + 1]
            part2 = part.reshape(2, nts[k] // 2, 2 * wf)
            r2 = r.reshape(ns[k] // 2, 2 * do)
            f, r = _epi_prep(part2, r2, Wk, Wr, bb.reshape(1, -1), ns[k],
                             do, with_u=(k > 0), n_tab_next=nts[k + 1])

    bt = batch[::16].reshape(-1, 1)
    return _epi_head(part, r, bt, fc1_W, fc1_b.reshape(1, -1), fc2_W,
                     fc2_b.reshape(1, -1), ns[4], dos[4], n_graphs)


# speedup vs baseline: 1.1444x; 1.1444x over previous
import jax
import jax.numpy as jnp
from jax import lax
from jax.experimental import pallas as pl
from jax.experimental.pallas import tpu as pltpu
from jax.experimental.pallas import tpu_sc as plsc

_NC = 2
_NS = 16
_NW = _NC * _NS
_B = 64
_XCOL = 16


def _elu(v):
    return jnp.where(v > 0, v, jnp.exp(v) - 1.0)


def _prep(h_in, Wk, Wr, b2d, n_rows, n_tab, pool):
    di, do = Wk.shape[1], Wk.shape[2]
    wf = 2 * do + _XCOL

    def body(h_ref, wk_ref, wr_ref, b_ref, f_ref, r_ref):
        hv = h_ref[...]
        if pool:
            hp = jnp.maximum(hv[:, :di], hv[:, di:])
        else:
            hp = hv
        w0 = wk_ref[0]
        wd = wk_ref[1] - w0
        zc = jnp.zeros((di, _XCOL), jnp.float32)
        wcat = jnp.concatenate([w0, zc, wd], axis=1)
        hp_pad = jnp.concatenate(
            [hp, jnp.zeros((n_tab - n_rows, di), jnp.float32)], axis=0)
        cols = lax.broadcasted_iota(jnp.int32, (n_tab, wf), 1)
        ones_col = (cols == do).astype(jnp.float32)
        f_ref[...] = jnp.dot(hp_pad, wcat,
                             preferred_element_type=jnp.float32) + ones_col
        r_ref[...] = jnp.dot(hp, wr_ref[...],
                             preferred_element_type=jnp.float32) + b_ref[...]

    return pl.pallas_call(
        body,
        out_shape=[
            jax.ShapeDtypeStruct((n_tab, wf), jnp.float32),
            jax.ShapeDtypeStruct((n_rows, do), jnp.float32),
        ],
    )(h_in, Wk, Wr, b2d)


def _sc_agg(f, srcp, dstp, up, zer, shift, n_tab, wf, do, bsz, n_real):
    with_u = up is not None
    epw = srcp.shape[0] // _NW
    nb = epw // bsz
    rpt = n_tab // _NS
    nbuf = 2 if 2 * bsz * wf * 4 <= 480000 else 1

    def body(*refs):
        i = 4 if with_u else 3
        f_h, src_h, dst_h = refs[0], refs[1], refs[2]
        u_h = refs[3] if with_u else None
        zer_h = refs[i]
        out = refs[i + 1]
        rows = refs[i + 2:i + 2 + nbuf]
        isrc = refs[i + 2 + nbuf:i + 2 + 2 * nbuf]
        idst = refs[i + 2 + 2 * nbuf:i + 2 + 3 * nbuf]
        i += 2 + 3 * nbuf
        uv = refs[i] if with_u else None
        i += 1 if with_u else 0
        acc = refs[i]
        gsem = refs[i + 1:i + 1 + nbuf]

        c = lax.axis_index("c")
        s = lax.axis_index("s")
        wid = s * _NC + c
        r0 = s * rpt
        pltpu.sync_copy(zer_h.at[pl.ds(r0, rpt)], acc.at[pl.ds(r0, rpt)])
        plsc.subcore_barrier()

        def load_idx(b, p):
            off = wid * epw + b * bsz
            pltpu.sync_copy(src_h.at[pl.ds(off, bsz)], isrc[p])
            pltpu.sync_copy(dst_h.at[pl.ds(off, bsz)], idst[p])
            if shift:
                for i2 in range(bsz // 16):
                    sl = pl.ds(i2 * 16, 16)
                    isrc[p][sl] = jnp.right_shift(isrc[p][sl], shift)
                    idst[p][sl] = jnp.right_shift(idst[p][sl], shift)

        if nbuf == 2:
            load_idx(0, 0)
            cp = pltpu.async_copy(f_h.at[isrc[0]], rows[0], gsem[0])
        else:
            def bbody1(b, carry):
                load_idx(b, 0)
                pltpu.async_copy(f_h.at[isrc[0]], rows[0], gsem[0]).wait()
                pltpu.sync_copy(rows[0], acc.at[idst[0]], add=True)
                return carry

            lax.fori_loop(0, nb, bbody1, 0)
            plsc.subcore_barrier()
            pltpu.sync_copy(acc.at[pl.ds(r0, rpt)], out.at[c, pl.ds(r0,
                                                                    rpt)])
            return

        def bbody(bi, carry):
            for p in range(2):
                b = bi * 2 + p
                q = 1 - p
                nxt = jnp.minimum(b + 1, nb - 1)
                load_idx(nxt, q)
                pltpu.async_copy(f_h.at[isrc[q]], rows[q], gsem[q])
                pltpu.make_async_copy(f_h.at[isrc[p]], rows[p],
                                      gsem[p]).wait()
                if with_u:
                    off = wid * epw + b * bsz
                    pltpu.sync_copy(u_h.at[pl.ds(off, bsz)], uv)
                    iot = lax.broadcasted_iota(jnp.int32, (16,), 0)
                    for ch in range(bsz // 16):
                        uc = uv[pl.ds(ch * 16, 16)]
                        uc = jnp.minimum(jnp.maximum(uc, 0.0), 1.0)
                        for e2 in range(16):
                            ue = jnp.sum(jnp.where(iot == e2, uc, 0.0))
                            e = ch * 16 + e2
                            for ci in range(do // 16):
                                slc = pl.ds(ci * 16, 16)
                                sld = pl.ds(do + _XCOL + ci * 16, 16)
                                rows[p][e, slc] = (rows[p][e, slc]
                                                   + ue * rows[p][e, sld])
                pltpu.sync_copy(rows[p], acc.at[idst[p]], add=True)
            return carry

        lax.fori_loop(0, nb // 2, bbody, 0)
        pltpu.make_async_copy(f_h.at[isrc[0]], rows[0], gsem[0]).wait()
        plsc.subcore_barrier()
        pltpu.sync_copy(acc.at[pl.ds(r0, rpt)], out.at[c, pl.ds(r0, rpt)])

    out_type = [jax.ShapeDtypeStruct((_NC, n_tab, wf), jnp.float32)]
    scratch = ([pltpu.VMEM((bsz, wf), jnp.float32)] * nbuf
               + [pltpu.VMEM((bsz,), jnp.int32)] * (2 * nbuf))
    if with_u:
        scratch.append(pltpu.VMEM((bsz,), jnp.float32))
    scratch += ([pltpu.VMEM_SHARED((n_tab, wf), jnp.float32)]
                + [pltpu.SemaphoreType.DMA] * nbuf)
    mesh = plsc.VectorSubcoreMesh(core_axis_name="c", subcore_axis_name="s",
                                  num_cores=_NC, num_subcores=_NS)
    fn = pl.kernel(
        body, out_type=out_type, mesh=mesh, scratch_types=scratch,
        compiler_params=pltpu.CompilerParams(needs_layout_passes=False,
                                             use_tc_tiling_on_sc=False))
    res = fn(f, srcp, dstp, up, zer) if with_u else fn(f, srcp, dstp, zer)
    if isinstance(res, (list, tuple)):
        res = res[0]
    return res


def _epi_h(p_ref, r_ref, n_rows, do, with_u):
    am = p_ref[0] + p_ref[1]
    am = am[:n_rows]
    deg = am[:, do:do + 1]
    agg = am[:, :do]
    if with_u:
        u = deg / jnp.maximum(jnp.max(deg), 1.0)
        agg = agg + u * am[:, do + _XCOL:2 * do + _XCOL]
    return _elu(agg / jnp.maximum(deg, 1.0) + r_ref[...])


def _epi_prep(part2, r2, Wk, Wr, b2d, n_rows, do, with_u, n_tab_next):
    di, do2 = Wk.shape[1], Wk.shape[2]
    wf = 2 * do + _XCOL
    wf2 = 2 * do2 + _XCOL
    n2 = n_rows // 2

    def body(p_ref, r_ref, wk_ref, wr_ref, b_ref, f_ref, r2_ref):
        am = p_ref[0] + p_ref[1]
        am = am[:n2]
        rv = r_ref[...]
        halves = []
        for o, ro in ((0, 0), (wf, do)):
            deg = am[:, o + do:o + do + 1]
            agg = am[:, o:o + do]
            if with_u:
                dsum = am[:, o + do + _XCOL:o + 2 * do + _XCOL]
                halves.append((agg, dsum, deg, rv[:, ro:ro + do]))
            else:
                halves.append((agg, None, deg, rv[:, ro:ro + do]))
        if with_u:
            maxdeg = jnp.maximum(jnp.max(halves[0][2]),
                                 jnp.max(halves[1][2]))
            maxdeg = jnp.maximum(maxdeg, 1.0)
        hs = []
        for agg, dsum, deg, rh in halves:
            if with_u:
                agg = agg + (deg / maxdeg) * dsum
            hs.append(_elu(agg / jnp.maximum(deg, 1.0) + rh))
        hp = jnp.maximum(hs[0], hs[1])
        w0 = wk_ref[0]
        wd = wk_ref[1] - w0
        zc = jnp.zeros((di, _XCOL), jnp.float32)
        wcat = jnp.concatenate([w0, zc, wd], axis=1)
        hp_pad = jnp.concatenate(
            [hp, jnp.zeros((n_tab_next - n2, di), jnp.float32)], axis=0)
        cols = lax.broadcasted_iota(jnp.int32, (n_tab_next, wf2), 1)
        ones_col = (cols == do2).astype(jnp.float32)
        f_ref[...] = jnp.dot(hp_pad, wcat,
                             preferred_element_type=jnp.float32) + ones_col
        r2_ref[...] = jnp.dot(hp, wr_ref[...],
                              preferred_element_type=jnp.float32) + b_ref[...]

    return pl.pallas_call(
        body,
        out_shape=[
            jax.ShapeDtypeStruct((n_tab_next, wf2), jnp.float32),
            jax.ShapeDtypeStruct((n2, do2), jnp.float32),
        ],
    )(part2, r2, Wk, Wr, b2d)


def _epi_head(part, r, bt, w1, b1, w2, b2, n_rows, do, n_graphs):
    def body(p_ref, r_ref, bt_ref, w1_ref, b1_ref, w2_ref, b2_ref, o_ref):
        hv = _epi_h(p_ref, r_ref, n_rows, do, with_u=True)
        btv = bt_ref[...]
        ninf = jnp.float32(-jnp.inf)
        gs = []
        for j in range(n_graphs):
            mj = jnp.where(btv == j, hv, ninf)
            gs.append(jnp.max(mj, axis=0, keepdims=True))
        g = jnp.concatenate(gs, axis=0)
        g = jnp.where(jnp.isfinite(g), g, 0.0)
        z = _elu(jnp.dot(g, w1_ref[...],
                         preferred_element_type=jnp.float32) + b1_ref[...])
        z2 = jnp.dot(z, w2_ref[...],
                     preferred_element_type=jnp.float32) + b2_ref[...]
        m = jnp.max(z2, axis=1, keepdims=True)
        lse = m + jnp.log(jnp.sum(jnp.exp(z2 - m), axis=1, keepdims=True))
        o_ref[...] = z2 - lse

    out_shape = jax.ShapeDtypeStruct((n_graphs, w2.shape[1]), jnp.float32)
    return pl.pallas_call(body, out_shape=out_shape)(part, r, bt, w1, b1,
                                                     w2, b2)


def kernel(x, edge_index, weight, batch, W1, Wr1, b1, W2, Wr2, b2, W3, Wr3,
           b3, W4, Wr4, b4, W5, Wr5, b5, fc1_W, fc1_b, fc2_W, fc2_b):
    n0 = x.shape[0]
    n_edges = edge_index.shape[1]
    n_graphs = 16

    bszs = [128, 128, 128, 80, 32]
    chunk = _NW * 128
    ne = ((n_edges + chunk - 1) // chunk) * chunk
    pad = ne - n_edges
    spread = 31
    ns = [n0 >> k for k in range(5)]
    base_pad = n0 + 16 * (spread - 1)
    nts = [((base_pad >> k) // 16 + 1) * 16 for k in range(5)]

    pidx = (n0 + 16 * (jnp.arange(pad, dtype=jnp.int32) % spread)).astype(
        jnp.int32)
    srcp = jnp.concatenate([edge_index[0], pidx])
    dstp = jnp.concatenate([edge_index[1], pidx])
    up = jnp.concatenate([weight[:, 0], jnp.zeros((pad,), jnp.float32)])

    layers = [(W1, Wr1, b1), (W2, Wr2, b2), (W3, Wr3, b3), (W4, Wr4, b4),
              (W5, Wr5, b5)]
    dos = [layers[k][0].shape[2] for k in range(5)]

    f, r = _prep(x, W1, Wr1, b1.reshape(1, -1), ns[0], nts[0], pool=False)
    for k in range(5):
        do = dos[k]
        wf = 2 * do + _XCOL
        zer = jnp.zeros((nts[k], wf), jnp.float32)
        part = _sc_agg(f, srcp, dstp, up if k == 0 else None, zer, k,
                       nts[k], wf, do, bszs[k], ns[k])
        if k < 4:
            Wk, Wr, bb = layers[k + 1]
            part2 = part.reshape(2, nts[k] // 2, 2 * wf)
            r2 = r.reshape(ns[k] // 2, 2 * do)
            f, r = _epi_prep(part2, r2, Wk, Wr, bb.reshape(1, -1), ns[k],
                             do, with_u=(k > 0), n_tab_next=nts[k + 1])

    bt = batch[::16].reshape(-1, 1)
    return _epi_head(part, r, bt, fc1_W, fc1_b.reshape(1, -1), fc2_W,
                     fc2_b.reshape(1, -1), ns[4], dos[4], n_graphs)
